# baseline (speedup 1.0000x reference)
import functools
import math

import jax
import jax.numpy as jnp
from jax import lax
from jax.experimental import pallas as pl
from jax.experimental.pallas import tpu as pltpu
from jax.experimental.pallas import tpu_sc as plsc

_info = plsc.get_sparse_core_info()
_NC, _NS, _L = _info.num_cores, _info.num_subcores, _info.num_lanes
_NW = _NC * _NS

_CHUNK = 128
_NBUF = 4


@functools.lru_cache(maxsize=None)
def _make_kernel(T, D, scale):
    dt = D // 8

    mesh = plsc.VectorSubcoreMesh(core_axis_name="c", subcore_axis_name="s")

    @functools.partial(
        pl.kernel,
        mesh=mesh,
        out_type=jax.ShapeDtypeStruct((T, dt, _NW, 8, _CHUNK), jnp.float32),
        scratch_types=[
            pltpu.VMEM((T, _CHUNK), jnp.int32),
            pltpu.VMEM((_NBUF, _CHUNK, D), jnp.float32),
            pltpu.VMEM((_NBUF, dt, 8, _CHUNK), jnp.float32),
        ]
        + [pltpu.SemaphoreType.DMA] * (2 * _NBUF + 1),
        compiler_params=pltpu.CompilerParams(use_tc_tiling_on_sc=False, needs_layout_passes=False),
    )
    def k(idx_hbm, table_hbm, out_hbm, idx_v, gbuf, sbuf, *sems):
        isem = sems[0]
        gsems = sems[1 : 1 + _NBUF]
        ssems = sems[1 + _NBUF :]
        wid = lax.axis_index("s") * _NC + lax.axis_index("c")

        pltpu.async_copy(idx_hbm.at[wid], idx_v, isem).wait()

        for b in range(_NBUF):
            pltpu.async_copy(table_hbm.at[idx_v.at[b]], gbuf.at[b], gsems[b])

        rowsel = lax.iota(jnp.int32, _L)

        def outer(c0, carry):
            for b in range(_NBUF):
                t = c0 * _NBUF + b
                pltpu.make_async_copy(
                    table_hbm.at[idx_v.at[t]], gbuf.at[b], gsems[b]
                ).wait()

                @pl.when(c0 > 0)
                def _():
                    pltpu.make_async_copy(
                        sbuf.at[b], out_hbm.at[t - _NBUF, :, wid], ssems[b]
                    ).wait()

                gb = gbuf.at[b]

                def trans_body(ir, acc):
                    i = ir // 8
                    r = ir - i * 8
                    d = ir
                    colsel = jnp.full((_L,), d, jnp.int32)
                    for kk in range(_CHUNK // _L):
                        v = plsc.load_gather(gb, [rowsel + (kk * _L), colsel])
                        sbuf[b, i, r, pl.ds(kk * _L, _L)] = v * scale
                    return acc

                lax.fori_loop(0, D, trans_body, 0, unroll=4)

                pltpu.async_copy(sbuf.at[b], out_hbm.at[t, :, wid], ssems[b])

                @pl.when(t + _NBUF < T)
                def _():
                    pltpu.async_copy(
                        table_hbm.at[idx_v.at[t + _NBUF]], gbuf.at[b], gsems[b]
                    )

            return carry

        lax.fori_loop(0, T // _NBUF, outer, 0)

        for b in range(_NBUF):
            t = T - _NBUF + b
            pltpu.make_async_copy(
                sbuf.at[b], out_hbm.at[t, :, wid], ssems[b]
            ).wait()

    return k


def kernel(x, lut):
    Bb, T = x.shape
    D = lut.shape[1]
    scale = float(math.sqrt(D))
    idx = (
        x.astype(jnp.int32)
        .T.reshape(T, _NW, _CHUNK)
        .transpose(1, 0, 2)
    )
    o5 = _make_kernel(T, D, scale)(idx, lut)
    out = o5.transpose(2, 4, 0, 1, 3).reshape(Bb, T, D)
    return out
